# baseline (speedup 1.0000x reference)
import functools

import jax
import jax.numpy as jnp
from jax import lax
from jax.experimental import pallas as pl
from jax.experimental.pallas import tpu as pltpu
from jax.experimental.pallas import tpu_sc as plsc

_R = 8
_K = 5
_TV = 4096
_SLOTS = 8
_NEG = float("-inf")
_BIGI = 2**31 - 1


def _tc_body(x_ref, emb_ref, oidx_ref, xred_ref, rv_ref, ri_ref, *, V):
    B, D = x_ref.shape
    TV = emb_ref.shape[0]
    G = D // _R
    t = pl.program_id(0)

    @pl.when(t == 0)
    def _init():
        x = x_ref[...]
        xred_ref[...] = jnp.sum(x.reshape(B, G, _R), axis=-1)
        rv_ref[...] = jnp.full((_SLOTS, B), _NEG, jnp.float32)
        ri_ref[...] = jnp.zeros((_SLOTS, B), jnp.int32)

    ered = jnp.sum(emb_ref[...].reshape(TV, G, _R), axis=-1)
    s = lax.dot_general(
        ered, xred_ref[...],
        (((1,), (1,)), ((), ())),
        preferred_element_type=jnp.float32,
    )
    rowi = lax.broadcasted_iota(jnp.int32, (TV, B), 0)
    s = jnp.where(t * TV + rowi < V, s, _NEG)

    tvs, tis = [], []
    for _ in range(_K):
        m = jnp.max(s, axis=0, keepdims=True)
        a = jnp.min(jnp.where(s == m, rowi, TV), axis=0, keepdims=True)
        tvs.append(m)
        tis.append(t * TV + a)
        s = jnp.where(rowi == a, _NEG, s)

    comb_v = jnp.concatenate(
        [rv_ref[...]] + tvs + [jnp.full((3, B), _NEG, jnp.float32)], axis=0)
    comb_i = jnp.concatenate(
        [ri_ref[...]] + tis + [jnp.zeros((3, B), jnp.int32)], axis=0)
    slot = lax.broadcasted_iota(jnp.int32, (16, B), 0)
    nvs, nis = [], []
    for _ in range(_K):
        m = jnp.max(comb_v, axis=0, keepdims=True)
        pos = jnp.min(jnp.where(comb_v == m, slot, 16), axis=0, keepdims=True)
        sel = slot == pos
        nvs.append(m)
        nis.append(jnp.min(jnp.where(sel, comb_i, _BIGI), axis=0, keepdims=True))
        comb_v = jnp.where(sel, _NEG, comb_v)
    rv_ref[...] = jnp.concatenate(
        nvs + [jnp.full((3, B), _NEG, jnp.float32)], axis=0)
    ri_ref[...] = jnp.concatenate(
        nis + [jnp.zeros((3, B), jnp.int32)], axis=0)

    @pl.when(t == pl.num_programs(0) - 1)
    def _emit():
        oidx_ref[...] = ri_ref[...]


def _stage1_topk(inputs, embeddings, interpret=False):
    B, D = inputs.shape
    V = embeddings.shape[0]
    nt = (V + _TV - 1) // _TV
    return pl.pallas_call(
        functools.partial(_tc_body, V=V),
        grid=(nt,),
        in_specs=[
            pl.BlockSpec((B, D), lambda t: (0, 0)),
            pl.BlockSpec((_TV, D), lambda t: (t, 0)),
        ],
        out_specs=pl.BlockSpec((_SLOTS, B), lambda t: (0, 0)),
        out_shape=jax.ShapeDtypeStruct((_SLOTS, B), jnp.int32),
        scratch_shapes=[
            pltpu.VMEM((B, D // _R), jnp.float32),
            pltpu.VMEM((_SLOTS, B), jnp.float32),
            pltpu.VMEM((_SLOTS, B), jnp.int32),
        ],
        interpret=interpret,
    )(inputs, embeddings)


def _sc_body(emb_hbm, x_hbm, idx_hbm, probs_hbm, oidx_hbm,
             idx_v, rows_a, rows_b, x_v, probs_v, oidx_v, sem,
             *, B, D, BPW):
    c = lax.axis_index("c")
    s = lax.axis_index("s")
    wid = s * 2 + c
    base_r = wid * BPW

    for k in range(_K):
        pltpu.sync_copy(idx_hbm.at[pl.ds(k * B + base_r, BPW)],
                        idx_v.at[pl.ds(k * BPW, BPW)])
    pltpu.sync_copy(x_hbm.at[pl.ds(base_r, BPW)], x_v)
    half = _K * BPW // 2
    cp_a = pltpu.async_copy(emb_hbm.at[idx_v.at[pl.ds(0, half)]], rows_a, sem)
    cp_b = pltpu.async_copy(emb_hbm.at[idx_v.at[pl.ds(half, half)]], rows_b, sem)
    cp_a.wait()
    cp_b.wait()

    lane = lax.iota(jnp.int32, 16)
    for g in range(BPW // 16):
        xrow = g * 16 + lane

        def dot_step(d, accs, xrow=xrow, g=g):
            dv = jnp.full((16,), 0, jnp.int32) + d
            xv = plsc.load_gather(x_v, [xrow, dv])
            out = []
            for k, acc in enumerate(accs):
                p = k * BPW + g * 16
                rows_g, off = (rows_a, p) if p < half else (rows_b, p - half)
                ev = plsc.load_gather(rows_g, [off + lane, dv])
                out.append(acc + xv * ev)
            return tuple(out)

        logits = lax.fori_loop(
            0, D, dot_step,
            tuple(jnp.zeros((16,), jnp.float32) for _ in range(_K)))

        m = logits[0]
        for k in range(1, _K):
            m = jnp.maximum(m, logits[k])
        z = jnp.zeros((16,), jnp.float32)
        for k in range(_K):
            z = z + jnp.exp(logits[k] - m)
        prob = 1.0 / z
        best = jnp.full((16,), _K - 1, jnp.int32)
        for k in range(_K - 2, -1, -1):
            best = jnp.where(logits[k] == m, k, best)
        fidx = plsc.load_gather(idx_v, [best * BPW + g * 16 + lane])
        probs_v[pl.ds(g * 16, 16)] = prob
        oidx_v[pl.ds(g * 16, 16)] = fidx

    pltpu.sync_copy(probs_v, probs_hbm.at[pl.ds(base_r, BPW)])
    pltpu.sync_copy(oidx_v, oidx_hbm.at[pl.ds(base_r, BPW)])


def _stage2_rescore(embeddings, inputs, idx_flat):
    B, D = inputs.shape
    BPW = B // 32
    half = _K * BPW // 2
    mesh = plsc.VectorSubcoreMesh(core_axis_name="c", subcore_axis_name="s")
    return pl.kernel(
        functools.partial(_sc_body, B=B, D=D, BPW=BPW),
        out_type=(
            jax.ShapeDtypeStruct((B,), jnp.float32),
            jax.ShapeDtypeStruct((B,), jnp.int32),
        ),
        mesh=mesh,
        compiler_params=pltpu.CompilerParams(needs_layout_passes=False),
        scratch_types=[
            pltpu.VMEM((_K * BPW,), jnp.int32),
            pltpu.VMEM((half, D), jnp.float32),
            pltpu.VMEM((half, D), jnp.float32),
            pltpu.VMEM((BPW, D), jnp.float32),
            pltpu.VMEM((BPW,), jnp.float32),
            pltpu.VMEM((BPW,), jnp.int32),
            pltpu.SemaphoreType.DMA,
        ],
    )(embeddings, inputs, idx_flat)


def kernel(inputs, embeddings):
    dims = inputs.shape
    d = dims[-1]
    x = inputs.reshape(-1, d)
    top_idx = _stage1_topk(x, embeddings)
    idx_flat = top_idx.reshape(-1)
    probs, indices = _stage2_rescore(embeddings, x, idx_flat)
    return probs.reshape(dims[:-1]), indices.reshape(dims[:-1])


# speedup vs baseline: 1.3005x; 1.3005x over previous
import functools

import jax
import jax.numpy as jnp
from jax import lax
from jax.experimental import pallas as pl
from jax.experimental.pallas import tpu as pltpu
from jax.experimental.pallas import tpu_sc as plsc

_R = 8
_K = 5
_TV = 1024
_SLOTS = 8
_NEG = float("-inf")
_BIGI = 2**31 - 1


def _tc_body(x_ref, emb_ref, oidx_ref, xred_ref, rv_ref, ri_ref, *, V):
    B, D = x_ref.shape
    TV = emb_ref.shape[0]
    G = D // _R
    t = pl.program_id(0)

    @pl.when(t == 0)
    def _init():
        x = x_ref[...]
        xred_ref[...] = jnp.sum(x.reshape(B, G, _R), axis=-1)
        rv_ref[...] = jnp.full((_SLOTS, B), _NEG, jnp.float32)
        ri_ref[...] = jnp.zeros((_SLOTS, B), jnp.int32)

    ered = jnp.sum(emb_ref[...].reshape(TV, G, _R), axis=-1)
    s = lax.dot_general(
        ered, xred_ref[...],
        (((1,), (1,)), ((), ())),
        preferred_element_type=jnp.float32,
    )
    rowi = lax.broadcasted_iota(jnp.int32, (TV, B), 0)
    s = jnp.where(t * TV + rowi < V, s, _NEG)

    tvs, tis = [], []
    for _ in range(_K):
        m = jnp.max(s, axis=0, keepdims=True)
        a = jnp.min(jnp.where(s == m, rowi, TV), axis=0, keepdims=True)
        tvs.append(m)
        tis.append(t * TV + a)
        s = jnp.where(rowi == a, _NEG, s)

    comb_v = jnp.concatenate(
        [rv_ref[...]] + tvs + [jnp.full((3, B), _NEG, jnp.float32)], axis=0)
    comb_i = jnp.concatenate(
        [ri_ref[...]] + tis + [jnp.zeros((3, B), jnp.int32)], axis=0)
    slot = lax.broadcasted_iota(jnp.int32, (16, B), 0)
    nvs, nis = [], []
    for _ in range(_K):
        m = jnp.max(comb_v, axis=0, keepdims=True)
        pos = jnp.min(jnp.where(comb_v == m, slot, 16), axis=0, keepdims=True)
        sel = slot == pos
        nvs.append(m)
        nis.append(jnp.min(jnp.where(sel, comb_i, _BIGI), axis=0, keepdims=True))
        comb_v = jnp.where(sel, _NEG, comb_v)
    rv_ref[...] = jnp.concatenate(
        nvs + [jnp.full((3, B), _NEG, jnp.float32)], axis=0)
    ri_ref[...] = jnp.concatenate(
        nis + [jnp.zeros((3, B), jnp.int32)], axis=0)

    @pl.when(t == pl.num_programs(0) - 1)
    def _emit():
        oidx_ref[...] = ri_ref[...]


def _stage1_topk(inputs, embeddings, interpret=False):
    B, D = inputs.shape
    V = embeddings.shape[0]
    nt = (V + _TV - 1) // _TV
    return pl.pallas_call(
        functools.partial(_tc_body, V=V),
        grid=(nt,),
        in_specs=[
            pl.BlockSpec((B, D), lambda t: (0, 0)),
            pl.BlockSpec((_TV, D), lambda t: (t, 0)),
        ],
        out_specs=pl.BlockSpec((_SLOTS, B), lambda t: (0, 0)),
        out_shape=jax.ShapeDtypeStruct((_SLOTS, B), jnp.int32),
        scratch_shapes=[
            pltpu.VMEM((B, D // _R), jnp.float32),
            pltpu.VMEM((_SLOTS, B), jnp.float32),
            pltpu.VMEM((_SLOTS, B), jnp.int32),
        ],
        interpret=interpret,
    )(inputs, embeddings)


def _sc_body(emb_hbm, x_hbm, idx_hbm, probs_hbm, oidx_hbm,
             idx_v, rows_a, rows_b, x_v, probs_v, oidx_v, sem,
             *, B, D, BPW):
    c = lax.axis_index("c")
    s = lax.axis_index("s")
    wid = s * 2 + c
    base_r = wid * BPW

    for k in range(_K):
        pltpu.sync_copy(idx_hbm.at[pl.ds(k * B + base_r, BPW)],
                        idx_v.at[pl.ds(k * BPW, BPW)])
    pltpu.sync_copy(x_hbm.at[pl.ds(base_r, BPW)], x_v)
    half = _K * BPW // 2
    cp_a = pltpu.async_copy(emb_hbm.at[idx_v.at[pl.ds(0, half)]], rows_a, sem)
    cp_b = pltpu.async_copy(emb_hbm.at[idx_v.at[pl.ds(half, half)]], rows_b, sem)
    cp_a.wait()
    cp_b.wait()

    lane = lax.iota(jnp.int32, 16)
    for g in range(BPW // 16):
        xrow = g * 16 + lane

        def dot_step(d, accs, xrow=xrow, g=g):
            dv = jnp.full((16,), 0, jnp.int32) + d
            xv = plsc.load_gather(x_v, [xrow, dv])
            out = []
            for k, acc in enumerate(accs):
                p = k * BPW + g * 16
                rows_g, off = (rows_a, p) if p < half else (rows_b, p - half)
                ev = plsc.load_gather(rows_g, [off + lane, dv])
                out.append(acc + xv * ev)
            return tuple(out)

        logits = lax.fori_loop(
            0, D, dot_step,
            tuple(jnp.zeros((16,), jnp.float32) for _ in range(_K)))

        m = logits[0]
        for k in range(1, _K):
            m = jnp.maximum(m, logits[k])
        z = jnp.zeros((16,), jnp.float32)
        for k in range(_K):
            z = z + jnp.exp(logits[k] - m)
        prob = 1.0 / z
        best = jnp.full((16,), _K - 1, jnp.int32)
        for k in range(_K - 2, -1, -1):
            best = jnp.where(logits[k] == m, k, best)
        fidx = plsc.load_gather(idx_v, [best * BPW + g * 16 + lane])
        probs_v[pl.ds(g * 16, 16)] = prob
        oidx_v[pl.ds(g * 16, 16)] = fidx

    pltpu.sync_copy(probs_v, probs_hbm.at[pl.ds(base_r, BPW)])
    pltpu.sync_copy(oidx_v, oidx_hbm.at[pl.ds(base_r, BPW)])


def _stage2_rescore(embeddings, inputs, idx_flat):
    B, D = inputs.shape
    BPW = B // 32
    half = _K * BPW // 2
    mesh = plsc.VectorSubcoreMesh(core_axis_name="c", subcore_axis_name="s")
    return pl.kernel(
        functools.partial(_sc_body, B=B, D=D, BPW=BPW),
        out_type=(
            jax.ShapeDtypeStruct((B,), jnp.float32),
            jax.ShapeDtypeStruct((B,), jnp.int32),
        ),
        mesh=mesh,
        compiler_params=pltpu.CompilerParams(needs_layout_passes=False),
        scratch_types=[
            pltpu.VMEM((_K * BPW,), jnp.int32),
            pltpu.VMEM((half, D), jnp.float32),
            pltpu.VMEM((half, D), jnp.float32),
            pltpu.VMEM((BPW, D), jnp.float32),
            pltpu.VMEM((BPW,), jnp.float32),
            pltpu.VMEM((BPW,), jnp.int32),
            pltpu.SemaphoreType.DMA,
        ],
    )(embeddings, inputs, idx_flat)


def kernel(inputs, embeddings):
    dims = inputs.shape
    d = dims[-1]
    x = inputs.reshape(-1, d)
    top_idx = _stage1_topk(x, embeddings)
    idx_flat = top_idx.reshape(-1)
    probs, indices = _stage2_rescore(embeddings, x, idx_flat)
    return probs.reshape(dims[:-1]), indices.reshape(dims[:-1])


# speedup vs baseline: 1.3175x; 1.0131x over previous
import functools

import jax
import jax.numpy as jnp
from jax import lax
from jax.experimental import pallas as pl
from jax.experimental.pallas import tpu as pltpu
from jax.experimental.pallas import tpu_sc as plsc

_R = 8
_K = 5
_TV = 2048
_SLOTS = 8
_NEG = float("-inf")
_BIGI = 2**31 - 1


def _tc_body(x_ref, emb_ref, oidx_ref, xred_ref, rv_ref, ri_ref, *, V):
    B, D = x_ref.shape
    TV = emb_ref.shape[0]
    G = D // _R
    t = pl.program_id(0)

    @pl.when(t == 0)
    def _init():
        x = x_ref[...]
        xred_ref[...] = jnp.sum(x.reshape(B, G, _R), axis=-1)
        rv_ref[...] = jnp.full((_SLOTS, B), _NEG, jnp.float32)
        ri_ref[...] = jnp.zeros((_SLOTS, B), jnp.int32)

    ered = jnp.sum(emb_ref[...].reshape(TV, G, _R), axis=-1)
    s = lax.dot_general(
        ered, xred_ref[...],
        (((1,), (1,)), ((), ())),
        preferred_element_type=jnp.float32,
    )
    rowi = lax.broadcasted_iota(jnp.int32, (TV, B), 0)
    s = jnp.where(rowi < V - t * TV, s, _NEG)

    tvs, tis = [], []
    for k in range(_K):
        m = jnp.max(s, axis=0, keepdims=True)
        a = jnp.min(jnp.where(s == m, rowi, TV), axis=0, keepdims=True)
        tvs.append(m)
        tis.append(t * TV + a)
        if k + 1 < _K:
            s = jnp.where(rowi == a, _NEG, s)

    comb_v = jnp.concatenate(
        [rv_ref[...]] + tvs + [jnp.full((3, B), _NEG, jnp.float32)], axis=0)
    comb_i = jnp.concatenate(
        [ri_ref[...]] + tis + [jnp.zeros((3, B), jnp.int32)], axis=0)
    slot = lax.broadcasted_iota(jnp.int32, (16, B), 0)
    nvs, nis = [], []
    for _ in range(_K):
        m = jnp.max(comb_v, axis=0, keepdims=True)
        pos = jnp.min(jnp.where(comb_v == m, slot, 16), axis=0, keepdims=True)
        sel = slot == pos
        nvs.append(m)
        nis.append(jnp.min(jnp.where(sel, comb_i, _BIGI), axis=0, keepdims=True))
        comb_v = jnp.where(sel, _NEG, comb_v)
    rv_ref[...] = jnp.concatenate(
        nvs + [jnp.full((3, B), _NEG, jnp.float32)], axis=0)
    ri_ref[...] = jnp.concatenate(
        nis + [jnp.zeros((3, B), jnp.int32)], axis=0)

    @pl.when(t == pl.num_programs(0) - 1)
    def _emit():
        oidx_ref[...] = ri_ref[...]


def _stage1_topk(inputs, embeddings, interpret=False):
    B, D = inputs.shape
    V = embeddings.shape[0]
    nt = (V + _TV - 1) // _TV
    return pl.pallas_call(
        functools.partial(_tc_body, V=V),
        grid=(nt,),
        in_specs=[
            pl.BlockSpec((B, D), lambda t: (0, 0)),
            pl.BlockSpec((_TV, D), lambda t: (t, 0)),
        ],
        out_specs=pl.BlockSpec((_SLOTS, B), lambda t: (0, 0)),
        out_shape=jax.ShapeDtypeStruct((_SLOTS, B), jnp.int32),
        scratch_shapes=[
            pltpu.VMEM((B, D // _R), jnp.float32),
            pltpu.VMEM((_SLOTS, B), jnp.float32),
            pltpu.VMEM((_SLOTS, B), jnp.int32),
        ],
        interpret=interpret,
    )(inputs, embeddings)


def _sc_body(emb_hbm, x_hbm, idx_hbm, probs_hbm, oidx_hbm,
             idx_v, rows_a, rows_b, x_v, probs_v, oidx_v, sem,
             *, B, D, BPW):
    c = lax.axis_index("c")
    s = lax.axis_index("s")
    wid = s * 2 + c
    base_r = wid * BPW

    for k in range(_K):
        pltpu.sync_copy(idx_hbm.at[pl.ds(k * B + base_r, BPW)],
                        idx_v.at[pl.ds(k * BPW, BPW)])
    pltpu.sync_copy(x_hbm.at[pl.ds(base_r, BPW)], x_v)
    half = _K * BPW // 2
    cp_a = pltpu.async_copy(emb_hbm.at[idx_v.at[pl.ds(0, half)]], rows_a, sem)
    cp_b = pltpu.async_copy(emb_hbm.at[idx_v.at[pl.ds(half, half)]], rows_b, sem)
    cp_a.wait()
    cp_b.wait()

    lane = lax.iota(jnp.int32, 16)
    for g in range(BPW // 16):
        xrow = g * 16 + lane

        def dot_step(d, accs, xrow=xrow, g=g):
            dv = jnp.full((16,), 0, jnp.int32) + d
            xv = plsc.load_gather(x_v, [xrow, dv])
            out = []
            for k, acc in enumerate(accs):
                p = k * BPW + g * 16
                rows_g, off = (rows_a, p) if p < half else (rows_b, p - half)
                ev = plsc.load_gather(rows_g, [off + lane, dv])
                out.append(acc + xv * ev)
            return tuple(out)

        logits = lax.fori_loop(
            0, D, dot_step,
            tuple(jnp.zeros((16,), jnp.float32) for _ in range(_K)))

        m = logits[0]
        for k in range(1, _K):
            m = jnp.maximum(m, logits[k])
        z = jnp.zeros((16,), jnp.float32)
        for k in range(_K):
            z = z + jnp.exp(logits[k] - m)
        prob = 1.0 / z
        best = jnp.full((16,), _K - 1, jnp.int32)
        for k in range(_K - 2, -1, -1):
            best = jnp.where(logits[k] == m, k, best)
        fidx = plsc.load_gather(idx_v, [best * BPW + g * 16 + lane])
        probs_v[pl.ds(g * 16, 16)] = prob
        oidx_v[pl.ds(g * 16, 16)] = fidx

    pltpu.sync_copy(probs_v, probs_hbm.at[pl.ds(base_r, BPW)])
    pltpu.sync_copy(oidx_v, oidx_hbm.at[pl.ds(base_r, BPW)])


def _stage2_rescore(embeddings, inputs, idx_flat):
    B, D = inputs.shape
    BPW = B // 32
    half = _K * BPW // 2
    mesh = plsc.VectorSubcoreMesh(core_axis_name="c", subcore_axis_name="s")
    return pl.kernel(
        functools.partial(_sc_body, B=B, D=D, BPW=BPW),
        out_type=(
            jax.ShapeDtypeStruct((B,), jnp.float32),
            jax.ShapeDtypeStruct((B,), jnp.int32),
        ),
        mesh=mesh,
        compiler_params=pltpu.CompilerParams(needs_layout_passes=False),
        scratch_types=[
            pltpu.VMEM((_K * BPW,), jnp.int32),
            pltpu.VMEM((half, D), jnp.float32),
            pltpu.VMEM((half, D), jnp.float32),
            pltpu.VMEM((BPW, D), jnp.float32),
            pltpu.VMEM((BPW,), jnp.float32),
            pltpu.VMEM((BPW,), jnp.int32),
            pltpu.SemaphoreType.DMA,
        ],
    )(embeddings, inputs, idx_flat)


def kernel(inputs, embeddings):
    dims = inputs.shape
    d = dims[-1]
    x = inputs.reshape(-1, d)
    top_idx = _stage1_topk(x, embeddings)
    idx_flat = top_idx.reshape(-1)
    probs, indices = _stage2_rescore(embeddings, x, idx_flat)
    return probs.reshape(dims[:-1]), indices.reshape(dims[:-1])
